# baseline (speedup 1.0000x reference)
import jax
import jax.numpy as jnp
from jax import lax
from jax.experimental import pallas as pl
from jax.experimental.pallas import tpu as pltpu
from jax.experimental.pallas import tpu_sc as plsc

B, C, H, W = 32, 3, 512, 512
P = 16
EMBED = 96
N = (H // P) * (W // P)
NW = 32
CHUNK = 16
NCHUNK = N // CHUNK
NSLICE = 4
NIMG = B // NSLICE
WPI = NW // NIMG
NCH_W = NCHUNK // WPI
SEG = C * P
RPP = 2 * SEG
NG = CHUNK * RPP // 128
KDIM = C * P * P
NKT = KDIM // 128
TAB_ROWS = B * C * H * W // 16


def _sc_gather(table, ys, xs, img_base):
    mesh = plsc.VectorSubcoreMesh(core_axis_name="c", subcore_axis_name="s")

    def body(table_hbm, ys_hbm, xs_hbm, out_hbm, ysv, xsv, ib0, ib1,
             stg0, stg1, ov0, ov1, semg0, semg1, semo0, semo1):
        w = lax.axis_index("s") * 2 + lax.axis_index("c")
        img_l = w // WPI
        cbase = (w % WPI) * NCH_W
        j16 = jax.lax.iota(jnp.int32, 16)
        iota512 = j16 * W
        tiota2 = j16 * 2

        def compute_idx(gi, ib):
            def pbody(p, _):
                n = gi * CHUNK + p
                ysp = ysv[pl.ds(n, 16)][0]
                xsp = xsv[pl.ds(n, 16)][0]
                s0 = (img_base + img_l) * (C * H * W) + ysp * W + xsp
                qbase = p * RPP
                for c in range(C):
                    sv = (s0 + c * (H * W)) + iota512
                    r0 = sv >> 4
                    r1 = jnp.minimum(r0 + 1, TAB_ROWS - 1)
                    qe = (qbase + c * 32) + tiota2
                    plsc.store_scatter(ib, [qe], r0)
                    plsc.store_scatter(ib, [qe + 1], r1)
                return ()
            lax.fori_loop(0, CHUNK, pbody, ())

        def fire(ib, stg, semg):
            for g in range(NG):
                pltpu.async_copy(table_hbm.at[ib.at[pl.ds(g * 128, 128)]],
                                 stg.at[pl.ds(g * 128, 128), :], semg)

        def drain_gathers(stg, semg):
            for g in range(NG):
                pltpu.make_async_copy(table_hbm.at[pl.ds(0, 128), :],
                                      stg.at[pl.ds(g * 128, 128), :],
                                      semg).wait()

        def realign(gi, stg, ov):
            def patch_body(p, _):
                xsp = xsv[pl.ds(gi * CHUNK + p, 16)][0]
                t = (xsp & 15) + j16
                rowv = (t >> 4) + p * RPP
                colv = t & 15
                rb = p >> 3
                r8 = p & 7
                for k in range(SEG):
                    vals = plsc.load_gather(stg, [rowv + (2 * k), colv])
                    ov[rb, k >> 3, r8, pl.ds((k & 7) * 16, 16)] = vals
                return ()
            lax.fori_loop(0, CHUNK, patch_body, ())

        def drain_out(ov, semo):
            pltpu.make_async_copy(ov, out_hbm.at[pl.ds(0, 2)], semo).wait()

        pltpu.sync_copy(ys_hbm.at[img_l], ysv.at[pl.ds(0, N)])
        pltpu.sync_copy(xs_hbm.at[img_l], xsv.at[pl.ds(0, N)])
        compute_idx(cbase, ib0)
        fire(ib0, stg0, semg0)

        def chunk_iter(ci, _):
            gi = cbase + ci

            def process(cur_ib, cur_stg, cur_ov, cur_semg, cur_semo,
                        nxt_ib, nxt_stg, nxt_semg):
                @pl.when(ci < NCH_W - 1)
                def _():
                    compute_idx(gi + 1, nxt_ib)
                    fire(nxt_ib, nxt_stg, nxt_semg)
                drain_gathers(cur_stg, cur_semg)

                @pl.when(ci >= 2)
                def _():
                    drain_out(cur_ov, cur_semo)
                realign(gi, cur_stg, cur_ov)
                pltpu.async_copy(
                    cur_ov, out_hbm.at[pl.ds(img_l * (N // 8) + gi * 2, 2)],
                    cur_semo)

            @pl.when(ci % 2 == 0)
            def _():
                process(ib0, stg0, ov0, semg0, semo0, ib1, stg1, semg1)

            @pl.when(ci % 2 == 1)
            def _():
                process(ib1, stg1, ov1, semg1, semo1, ib0, stg0, semg0)
            return ()

        lax.fori_loop(0, NCH_W, chunk_iter, ())
        drain_out(ov0, semo0)
        drain_out(ov1, semo1)

    run = pl.kernel(
        body,
        out_type=jax.ShapeDtypeStruct((NIMG * N // 8, NKT, 8, 128),
                                      jnp.float32),
        mesh=mesh,
        scratch_types=[
            pltpu.VMEM((N + 16,), jnp.int32),
            pltpu.VMEM((N + 16,), jnp.int32),
            pltpu.VMEM((CHUNK * RPP,), jnp.int32),
            pltpu.VMEM((CHUNK * RPP,), jnp.int32),
            pltpu.VMEM((CHUNK * RPP, 16), jnp.float32),
            pltpu.VMEM((CHUNK * RPP, 16), jnp.float32),
            pltpu.VMEM((2, NKT, 8, 128), jnp.float32),
            pltpu.VMEM((2, NKT, 8, 128), jnp.float32),
            pltpu.SemaphoreType.DMA,
            pltpu.SemaphoreType.DMA,
            pltpu.SemaphoreType.DMA,
            pltpu.SemaphoreType.DMA,
        ],
        compiler_params=pltpu.CompilerParams(
            needs_layout_passes=False, use_tc_tiling_on_sc=False),
    )
    return run(table, ys, xs)


def _tc_project(p4, Wp3, bp2):
    M = NIMG * N
    BM = 512

    def mm_body(p_ref, w_ref, b_ref, o_ref):
        acc = jnp.zeros((BM, EMBED), jnp.float32)
        for c in range(NKT):
            xc = p_ref[:, c].reshape(BM, 128)
            acc = acc + jnp.dot(xc, w_ref[c],
                                preferred_element_type=jnp.float32)
        o_ref[...] = acc + b_ref[...]

    return pl.pallas_call(
        mm_body,
        grid=(M // BM,),
        in_specs=[
            pl.BlockSpec((BM // 8, NKT, 8, 128), lambda i: (i, 0, 0, 0)),
            pl.BlockSpec((NKT, 128, EMBED), lambda i: (0, 0, 0)),
            pl.BlockSpec((1, EMBED), lambda i: (0, 0)),
        ],
        out_specs=pl.BlockSpec((BM, EMBED), lambda i: (i, 0)),
        out_shape=jax.ShapeDtypeStruct((M, EMBED), jnp.float32),
    )(p4, Wp3, bp2)


def kernel(x, ys, xs, Wp, bp):
    ys = ys.astype(jnp.int32)
    xs = xs.astype(jnp.int32)
    table = x.reshape(TAB_ROWS, 16)
    Wp3 = Wp.reshape(NKT, 128, EMBED)
    bp2 = bp.reshape(1, EMBED)
    toks = []
    for sl in range(NSLICE):
        i0 = sl * NIMG
        patches = _sc_gather(table, ys[i0:i0 + NIMG], xs[i0:i0 + NIMG], i0)
        toks.append(_tc_project(patches, Wp3, bp2))
    tokens = jnp.concatenate(toks, axis=0).reshape(B, N, EMBED)
    pos = jnp.stack([ys, xs], axis=-1)
    return (tokens, pos)


# speedup vs baseline: 1.1635x; 1.1635x over previous
import jax
import jax.numpy as jnp
from jax import lax
from jax.experimental import pallas as pl
from jax.experimental.pallas import tpu as pltpu
from jax.experimental.pallas import tpu_sc as plsc

B, C, H, W = 32, 3, 512, 512
P = 16
EMBED = 96
N = (H // P) * (W // P)
NW = 32
CHUNK = 16
NCHUNK = N // CHUNK
NSLICE = 2
NIMG = B // NSLICE
WPI = NW // NIMG
NCH_W = NCHUNK // WPI
SEG = C * P
RPP = 2 * SEG
NG = CHUNK * RPP // 128
KDIM = C * P * P
NKT = KDIM // 128
TAB_ROWS = B * C * H * W // 16


def _sc_gather(table, ys, xs, img_base):
    mesh = plsc.VectorSubcoreMesh(core_axis_name="c", subcore_axis_name="s")

    def body(table_hbm, ys_hbm, xs_hbm, out_hbm, ysv, xsv, ib0, ib1,
             stg0, stg1, ov0, ov1, semg0, semg1, semo0, semo1):
        w = lax.axis_index("s") * 2 + lax.axis_index("c")
        img_l = w // WPI
        cbase = (w % WPI) * NCH_W
        j16 = jax.lax.iota(jnp.int32, 16)
        iota512 = j16 * W
        tiota2 = j16 * 2

        def compute_idx(gi, ib):
            def pbody(p, _):
                n = gi * CHUNK + p
                ysp = ysv[pl.ds(n, 16)][0]
                xsp = xsv[pl.ds(n, 16)][0]
                s0 = (img_base + img_l) * (C * H * W) + ysp * W + xsp
                qbase = p * RPP
                for c in range(C):
                    sv = (s0 + c * (H * W)) + iota512
                    r0 = sv >> 4
                    r1 = jnp.minimum(r0 + 1, TAB_ROWS - 1)
                    qe = (qbase + c * 32) + tiota2
                    plsc.store_scatter(ib, [qe], r0)
                    plsc.store_scatter(ib, [qe + 1], r1)
                return ()
            lax.fori_loop(0, CHUNK, pbody, ())

        def fire(ib, stg, semg):
            for g in range(NG):
                pltpu.async_copy(table_hbm.at[ib.at[pl.ds(g * 128, 128)]],
                                 stg.at[pl.ds(g * 128, 128), :], semg)

        def drain_gathers(stg, semg):
            for g in range(NG):
                pltpu.make_async_copy(table_hbm.at[pl.ds(0, 128), :],
                                      stg.at[pl.ds(g * 128, 128), :],
                                      semg).wait()

        def realign(gi, stg, ov):
            @plsc.parallel_loop(0, CHUNK, 1, unroll=2)
            def patch_body(p):
                xsp = xsv[pl.ds(gi * CHUNK + p, 16)][0]
                t = (xsp & 15) + j16
                rowv = (t >> 4) + p * RPP
                colv = t & 15
                rb = p >> 3
                r8 = p & 7
                for k in range(SEG):
                    vals = plsc.load_gather(stg, [rowv + (2 * k), colv])
                    ov[rb, k >> 3, r8, pl.ds((k & 7) * 16, 16)] = vals

        def drain_out(ov, semo):
            pltpu.make_async_copy(ov, out_hbm.at[pl.ds(0, 2)], semo).wait()

        pltpu.sync_copy(ys_hbm.at[img_l], ysv.at[pl.ds(0, N)])
        pltpu.sync_copy(xs_hbm.at[img_l], xsv.at[pl.ds(0, N)])
        compute_idx(cbase, ib0)
        fire(ib0, stg0, semg0)

        def chunk_iter(ci, _):
            gi = cbase + ci

            def process(cur_ib, cur_stg, cur_ov, cur_semg, cur_semo,
                        nxt_ib, nxt_stg, nxt_semg):
                @pl.when(ci < NCH_W - 1)
                def _():
                    compute_idx(gi + 1, nxt_ib)
                    fire(nxt_ib, nxt_stg, nxt_semg)
                drain_gathers(cur_stg, cur_semg)

                @pl.when(ci >= 2)
                def _():
                    drain_out(cur_ov, cur_semo)
                realign(gi, cur_stg, cur_ov)
                pltpu.async_copy(
                    cur_ov, out_hbm.at[pl.ds(img_l * (N // 8) + gi * 2, 2)],
                    cur_semo)

            @pl.when(ci % 2 == 0)
            def _():
                process(ib0, stg0, ov0, semg0, semo0, ib1, stg1, semg1)

            @pl.when(ci % 2 == 1)
            def _():
                process(ib1, stg1, ov1, semg1, semo1, ib0, stg0, semg0)
            return ()

        lax.fori_loop(0, NCH_W, chunk_iter, ())
        drain_out(ov0, semo0)
        drain_out(ov1, semo1)

    run = pl.kernel(
        body,
        out_type=jax.ShapeDtypeStruct((NIMG * N // 8, NKT, 8, 128),
                                      jnp.float32),
        mesh=mesh,
        scratch_types=[
            pltpu.VMEM((N + 16,), jnp.int32),
            pltpu.VMEM((N + 16,), jnp.int32),
            pltpu.VMEM((CHUNK * RPP,), jnp.int32),
            pltpu.VMEM((CHUNK * RPP,), jnp.int32),
            pltpu.VMEM((CHUNK * RPP, 16), jnp.float32),
            pltpu.VMEM((CHUNK * RPP, 16), jnp.float32),
            pltpu.VMEM((2, NKT, 8, 128), jnp.float32),
            pltpu.VMEM((2, NKT, 8, 128), jnp.float32),
            pltpu.SemaphoreType.DMA,
            pltpu.SemaphoreType.DMA,
            pltpu.SemaphoreType.DMA,
            pltpu.SemaphoreType.DMA,
        ],
        compiler_params=pltpu.CompilerParams(
            needs_layout_passes=False, use_tc_tiling_on_sc=False),
    )
    return run(table, ys, xs)


def _tc_project(p4, Wp3, bp2):
    M = NIMG * N
    BM = 512

    def mm_body(p_ref, w_ref, b_ref, o_ref):
        acc = jnp.zeros((BM, EMBED), jnp.float32)
        for c in range(NKT):
            xc = p_ref[:, c].reshape(BM, 128)
            acc = acc + jnp.dot(xc, w_ref[c],
                                preferred_element_type=jnp.float32)
        o_ref[...] = acc + b_ref[...]

    return pl.pallas_call(
        mm_body,
        grid=(M // BM,),
        in_specs=[
            pl.BlockSpec((BM // 8, NKT, 8, 128), lambda i: (i, 0, 0, 0)),
            pl.BlockSpec((NKT, 128, EMBED), lambda i: (0, 0, 0)),
            pl.BlockSpec((1, EMBED), lambda i: (0, 0)),
        ],
        out_specs=pl.BlockSpec((BM, EMBED), lambda i: (i, 0)),
        out_shape=jax.ShapeDtypeStruct((M, EMBED), jnp.float32),
    )(p4, Wp3, bp2)


def kernel(x, ys, xs, Wp, bp):
    ys = ys.astype(jnp.int32)
    xs = xs.astype(jnp.int32)
    table = x.reshape(TAB_ROWS, 16)
    Wp3 = Wp.reshape(NKT, 128, EMBED)
    bp2 = bp.reshape(1, EMBED)
    toks = []
    for sl in range(NSLICE):
        i0 = sl * NIMG
        patches = _sc_gather(table, ys[i0:i0 + NIMG], xs[i0:i0 + NIMG], i0)
        toks.append(_tc_project(patches, Wp3, bp2))
    tokens = jnp.concatenate(toks, axis=0).reshape(B, N, EMBED)
    pos = jnp.stack([ys, xs], axis=-1)
    return (tokens, pos)


# speedup vs baseline: 1.2819x; 1.1018x over previous
import jax
import jax.numpy as jnp
from jax import lax
from jax.experimental import pallas as pl
from jax.experimental.pallas import tpu as pltpu
from jax.experimental.pallas import tpu_sc as plsc

B, C, H, W = 32, 3, 512, 512
P = 16
EMBED = 96
N = (H // P) * (W // P)
NW = 32
CHUNK = 16
NCHUNK = N // CHUNK
NSLICE = 2
NIMG = B // NSLICE
WPI = NW // NIMG
NCH_W = NCHUNK // WPI
SEG = C * P
RPP = 2 * SEG
NG = CHUNK * RPP // 128
KDIM = C * P * P
NKT = KDIM // 128
TAB_ROWS = B * C * H * W // 16


def _sc_gather(table, ys, xs, img_base):
    mesh = plsc.VectorSubcoreMesh(core_axis_name="c", subcore_axis_name="s")

    def body(table_hbm, ys_hbm, xs_hbm, out_hbm, ysv, xsv, ib0, ib1,
             stg0, stg1, ov0, ov1, semg0, semg1, semo0, semo1):
        w = lax.axis_index("s") * 2 + lax.axis_index("c")
        img_l = w // WPI
        cbase = (w % WPI) * NCH_W
        j16 = jax.lax.iota(jnp.int32, 16)
        iota512 = j16 * W
        tiota2 = j16 * 2

        def compute_idx(gi, ib):
            @plsc.parallel_loop(0, CHUNK, 1, unroll=2)
            def pbody(p):
                n = gi * CHUNK + p
                ysp = ysv[pl.ds(n, 16)][0]
                xsp = xsv[pl.ds(n, 16)][0]
                s0 = (img_base + img_l) * (C * H * W) + ysp * W + xsp
                qbase = p * RPP
                for c in range(C):
                    sv = (s0 + c * (H * W)) + iota512
                    r0 = sv >> 4
                    r1 = jnp.minimum(r0 + 1, TAB_ROWS - 1)
                    qe = (qbase + c * 32) + tiota2
                    plsc.store_scatter(ib, [qe], r0)
                    plsc.store_scatter(ib, [qe + 1], r1)

        def fire(ib, stg, semg):
            for g in range(NG):
                pltpu.async_copy(table_hbm.at[ib.at[pl.ds(g * 128, 128)]],
                                 stg.at[pl.ds(g * 128, 128), :], semg)

        def drain_gathers(stg, semg):
            for g in range(NG):
                pltpu.make_async_copy(table_hbm.at[pl.ds(0, 128), :],
                                      stg.at[pl.ds(g * 128, 128), :],
                                      semg).wait()

        def realign(gi, stg, ov):
            @plsc.parallel_loop(0, CHUNK, 1, unroll=4)
            def patch_body(p):
                xsp = xsv[pl.ds(gi * CHUNK + p, 16)][0]
                t = (xsp & 15) + j16
                rowv = (t >> 4) + p * RPP
                colv = t & 15
                rb = p >> 3
                r8 = p & 7
                for k in range(SEG):
                    vals = plsc.load_gather(stg, [rowv + (2 * k), colv])
                    ov[rb, k >> 3, r8, pl.ds((k & 7) * 16, 16)] = vals

        def drain_out(ov, semo):
            pltpu.make_async_copy(ov, out_hbm.at[pl.ds(0, 2)], semo).wait()

        pltpu.sync_copy(ys_hbm.at[img_l], ysv.at[pl.ds(0, N)])
        pltpu.sync_copy(xs_hbm.at[img_l], xsv.at[pl.ds(0, N)])
        compute_idx(cbase, ib0)
        fire(ib0, stg0, semg0)

        def chunk_iter(ci, _):
            gi = cbase + ci

            def process(cur_ib, cur_stg, cur_ov, cur_semg, cur_semo,
                        nxt_ib, nxt_stg, nxt_semg):
                @pl.when(ci < NCH_W - 1)
                def _():
                    compute_idx(gi + 1, nxt_ib)
                    fire(nxt_ib, nxt_stg, nxt_semg)
                drain_gathers(cur_stg, cur_semg)

                @pl.when(ci >= 2)
                def _():
                    drain_out(cur_ov, cur_semo)
                realign(gi, cur_stg, cur_ov)
                pltpu.async_copy(
                    cur_ov, out_hbm.at[pl.ds(img_l * (N // 8) + gi * 2, 2)],
                    cur_semo)

            @pl.when(ci % 2 == 0)
            def _():
                process(ib0, stg0, ov0, semg0, semo0, ib1, stg1, semg1)

            @pl.when(ci % 2 == 1)
            def _():
                process(ib1, stg1, ov1, semg1, semo1, ib0, stg0, semg0)
            return ()

        lax.fori_loop(0, NCH_W, chunk_iter, ())
        drain_out(ov0, semo0)
        drain_out(ov1, semo1)

    run = pl.kernel(
        body,
        out_type=jax.ShapeDtypeStruct((NIMG * N // 8, NKT, 8, 128),
                                      jnp.float32),
        mesh=mesh,
        scratch_types=[
            pltpu.VMEM((N + 16,), jnp.int32),
            pltpu.VMEM((N + 16,), jnp.int32),
            pltpu.VMEM((CHUNK * RPP,), jnp.int32),
            pltpu.VMEM((CHUNK * RPP,), jnp.int32),
            pltpu.VMEM((CHUNK * RPP, 16), jnp.float32),
            pltpu.VMEM((CHUNK * RPP, 16), jnp.float32),
            pltpu.VMEM((2, NKT, 8, 128), jnp.float32),
            pltpu.VMEM((2, NKT, 8, 128), jnp.float32),
            pltpu.SemaphoreType.DMA,
            pltpu.SemaphoreType.DMA,
            pltpu.SemaphoreType.DMA,
            pltpu.SemaphoreType.DMA,
        ],
        compiler_params=pltpu.CompilerParams(
            needs_layout_passes=False, use_tc_tiling_on_sc=False),
    )
    return run(table, ys, xs)


def _tc_project(p4, Wp3, bp2):
    M = NIMG * N
    BM = 512

    def mm_body(p_ref, w_ref, b_ref, o_ref):
        acc = jnp.zeros((BM, EMBED), jnp.float32)
        for c in range(NKT):
            xc = p_ref[:, c].reshape(BM, 128)
            acc = acc + jnp.dot(xc, w_ref[c],
                                preferred_element_type=jnp.float32)
        o_ref[...] = acc + b_ref[...]

    return pl.pallas_call(
        mm_body,
        grid=(M // BM,),
        in_specs=[
            pl.BlockSpec((BM // 8, NKT, 8, 128), lambda i: (i, 0, 0, 0)),
            pl.BlockSpec((NKT, 128, EMBED), lambda i: (0, 0, 0)),
            pl.BlockSpec((1, EMBED), lambda i: (0, 0)),
        ],
        out_specs=pl.BlockSpec((BM, EMBED), lambda i: (i, 0)),
        out_shape=jax.ShapeDtypeStruct((M, EMBED), jnp.float32),
    )(p4, Wp3, bp2)


def kernel(x, ys, xs, Wp, bp):
    ys = ys.astype(jnp.int32)
    xs = xs.astype(jnp.int32)
    table = x.reshape(TAB_ROWS, 16)
    Wp3 = Wp.reshape(NKT, 128, EMBED)
    bp2 = bp.reshape(1, EMBED)
    toks = []
    for sl in range(NSLICE):
        i0 = sl * NIMG
        patches = _sc_gather(table, ys[i0:i0 + NIMG], xs[i0:i0 + NIMG], i0)
        toks.append(_tc_project(patches, Wp3, bp2))
    tokens = jnp.concatenate(toks, axis=0).reshape(B, N, EMBED)
    pos = jnp.stack([ys, xs], axis=-1)
    return (tokens, pos)
